# baseline (speedup 1.0000x reference)
import functools

import jax
import jax.numpy as jnp
from jax import lax
from jax.experimental import pallas as pl
from jax.experimental.pallas import tpu as pltpu
from jax.experimental.pallas import tpu_sc as plsc

D = 32
BW = 512
LANES = 16


@functools.lru_cache(maxsize=None)
def _make_gather(batch: int, hist: int):
    info = plsc.get_sparse_core_info()
    nc, ns = info.num_cores, info.num_subcores
    nw = nc * ns
    assert batch // nw == BW and hist % 2 == 0
    n_pairs = hist // 2
    n_dtiles = D // 8
    tile_words = BW // 128 * 8 * 128
    dt_stride = (batch // 128) * 8 * 128
    h_stride = n_dtiles * dt_stride
    out_words = hist * h_stride

    mesh = plsc.VectorSubcoreMesh(core_axis_name="c", subcore_axis_name="s")

    @functools.partial(
        pl.kernel,
        mesh=mesh,
        out_type=jax.ShapeDtypeStruct((out_words,), jnp.float32),
        scratch_types=[
            pltpu.VMEM((BW * hist,), jnp.int32),
            pltpu.VMEM((BW,), jnp.int32),
            pltpu.VMEM((BW,), jnp.int32),
            pltpu.VMEM((BW, D), jnp.float32),
            pltpu.VMEM((BW, D), jnp.float32),
            pltpu.VMEM((tile_words * n_dtiles,), jnp.float32),
            pltpu.VMEM((tile_words * n_dtiles,), jnp.float32),
            pltpu.SemaphoreType.DMA,
            pltpu.SemaphoreType.DMA,
            pltpu.SemaphoreType.DMA,
            pltpu.SemaphoreType.DMA,
        ],
        compiler_params=pltpu.CompilerParams(
            use_tc_tiling_on_sc=False, needs_layout_passes=False
        ),
    )
    def k(idx_hbm, table_hbm, out_hbm, idx_blk, idx_a, idx_b, rows_a, rows_b,
          tiles_a, tiles_b, gsa, gsb, ssa, ssb):
        wid = lax.axis_index("s") * nc + lax.axis_index("c")
        lane = lax.iota(jnp.int32, LANES)
        lane_h = lane * hist
        out_base = wid * tile_words

        pltpu.sync_copy(idx_hbm.at[pl.ds(wid * (BW * hist), BW * hist)],
                        idx_blk)

        def repack(h, idxh):
            @plsc.parallel_loop(0, BW // LANES, unroll=8)
            def per_grp(g):
                vec = plsc.load_gather(idx_blk, [lane_h + (g * (LANES * hist) + h)])
                idxh[pl.ds(g * LANES, LANES)] = vec

        repack(0, idx_a)
        repack(1, idx_b)
        pltpu.async_copy(table_hbm.at[idx_a], rows_a, gsa)
        pltpu.async_copy(table_hbm.at[idx_b], rows_b, gsb)

        def transpose(rows, tiles):
            @plsc.parallel_loop(0, BW // LANES, unroll=8)
            def per_seg(t):
                dst_base = (t // 8) * 1024 + (t % 8) * LANES
                bvec = lane + t * LANES
                for d in range(D):
                    vec = plsc.load_gather(
                        rows, [bvec, jnp.full((LANES,), d, jnp.int32)]
                    )
                    tiles[pl.ds(dst_base + (d // 8) * 4096 + (d % 8) * 128,
                                LANES)] = vec

        def pair(j, carry):
            for s, idxh, rows, tiles, gs, ss in (
                (0, idx_a, rows_a, tiles_a, gsa, ssa),
                (1, idx_b, rows_b, tiles_b, gsb, ssb),
            ):
                h = 2 * j + s
                pltpu.make_async_copy(
                    table_hbm.at[idxh], rows, gs
                ).wait()

                @pl.when(j > 0)
                def _drain():
                    for dt in range(n_dtiles):
                        pltpu.make_async_copy(
                            tiles.at[pl.ds(dt * tile_words, tile_words)],
                            out_hbm.at[pl.ds(out_base, tile_words)],
                            ss,
                        ).wait()

                transpose(rows, tiles)
                for dt in range(n_dtiles):
                    pltpu.async_copy(
                        tiles.at[pl.ds(dt * tile_words, tile_words)],
                        out_hbm.at[
                            pl.ds(h * h_stride + dt * dt_stride + out_base,
                                  tile_words)
                        ],
                        ss,
                    )

                @pl.when(j < n_pairs - 1)
                def _next():
                    repack(h + 2, idxh)
                    pltpu.async_copy(table_hbm.at[idxh], rows, gs)

            return carry

        lax.fori_loop(0, n_pairs, pair, 0)
        for tiles, ss in ((tiles_a, ssa), (tiles_b, ssb)):
            for dt in range(n_dtiles):
                pltpu.make_async_copy(
                    tiles.at[pl.ds(dt * tile_words, tile_words)],
                    out_hbm.at[pl.ds(out_base, tile_words)],
                    ss,
                ).wait()

    return k


def kernel(token_ids, weight):
    batch, hist = token_ids.shape
    flat_idx = token_ids.reshape(batch * hist).astype(jnp.int32)
    outbuf = _make_gather(batch, hist)(flat_idx, weight)
    out = (
        outbuf.reshape(hist, D // 8, batch // 128, 8, 128)
        .transpose(2, 4, 0, 1, 3)
        .reshape(batch, hist, D)
    )
    return out


# speedup vs baseline: 1.0457x; 1.0457x over previous
import functools

import jax
import jax.numpy as jnp
from jax import lax
from jax.experimental import pallas as pl
from jax.experimental.pallas import tpu as pltpu
from jax.experimental.pallas import tpu_sc as plsc

D = 32
BW = 512
LANES = 16


@functools.lru_cache(maxsize=None)
def _make_gather(batch: int, hist: int):
    info = plsc.get_sparse_core_info()
    nc, ns = info.num_cores, info.num_subcores
    nw = nc * ns
    assert batch // nw == BW and hist % 2 == 0
    n_pairs = hist // 2
    n_dtiles = D // 8
    tile_words = BW // 128 * 8 * 128
    dt_stride = (batch // 128) * 8 * 128
    h_stride = n_dtiles * dt_stride
    out_words = hist * h_stride

    mesh = plsc.VectorSubcoreMesh(core_axis_name="c", subcore_axis_name="s")

    @functools.partial(
        pl.kernel,
        mesh=mesh,
        out_type=jax.ShapeDtypeStruct((out_words,), jnp.float32),
        scratch_types=[
            pltpu.VMEM((BW * hist,), jnp.int32),
            pltpu.VMEM((BW,), jnp.int32),
            pltpu.VMEM((BW,), jnp.int32),
            pltpu.VMEM((BW, D), jnp.float32),
            pltpu.VMEM((BW, D), jnp.float32),
            pltpu.VMEM((tile_words * n_dtiles,), jnp.float32),
            pltpu.VMEM((tile_words * n_dtiles,), jnp.float32),
            pltpu.SemaphoreType.DMA,
            pltpu.SemaphoreType.DMA,
            pltpu.SemaphoreType.DMA,
            pltpu.SemaphoreType.DMA,
        ],
        compiler_params=pltpu.CompilerParams(
            use_tc_tiling_on_sc=False, needs_layout_passes=False
        ),
    )
    def k(idx_hbm, table_hbm, out_hbm, idx_blk, idx_a, idx_b,
          rows_a, rows_b, tiles_a, tiles_b, gsa, gsb, ssa, ssb):
        wid = lax.axis_index("s") * nc + lax.axis_index("c")
        lane = lax.iota(jnp.int32, LANES)
        lane_h = lane * hist
        out_base = wid * tile_words

        pltpu.sync_copy(idx_hbm.at[pl.ds(wid * (BW * hist), BW * hist)],
                        idx_blk)

        def repack(h, idxh):
            @plsc.parallel_loop(0, BW // LANES, unroll=4)
            def per_grp(g):
                vec = plsc.load_gather(idx_blk, [lane_h + (g * (LANES * hist) + h)])
                idxh[pl.ds(g * LANES, LANES)] = vec

        table2 = table_hbm

        repack(0, idx_a)
        repack(1, idx_b)
        pltpu.async_copy(table2.at[idx_a], rows_a, gsa)
        pltpu.async_copy(table2.at[idx_b], rows_b, gsb)

        def transpose(rows, tiles):
            @plsc.parallel_loop(0, BW // LANES, unroll=4)
            def per_seg(t):
                dst_base = (t // 8) * 1024 + (t % 8) * LANES
                bvec = lane + t * LANES
                for d in range(D):
                    vec = plsc.load_gather(
                        rows, [bvec, jnp.full((LANES,), d, jnp.int32)]
                    )
                    tiles[pl.ds(dst_base + (d // 8) * 4096 + (d % 8) * 128,
                                LANES)] = vec

        def pair(j, carry):
            for s, idxh, rows, tiles, gs, ss in (
                (0, idx_a, rows_a, tiles_a, gsa, ssa),
                (1, idx_b, rows_b, tiles_b, gsb, ssb),
            ):
                h = 2 * j + s
                pltpu.make_async_copy(
                    table2.at[idxh], rows, gs
                ).wait()

                @pl.when(j > 0)
                def _drain():
                    for dt in range(n_dtiles):
                        pltpu.make_async_copy(
                            tiles.at[pl.ds(dt * tile_words, tile_words)],
                            out_hbm.at[pl.ds(out_base, tile_words)],
                            ss,
                        ).wait()

                transpose(rows, tiles)
                for dt in range(n_dtiles):
                    pltpu.async_copy(
                        tiles.at[pl.ds(dt * tile_words, tile_words)],
                        out_hbm.at[
                            pl.ds(h * h_stride + dt * dt_stride + out_base,
                                  tile_words)
                        ],
                        ss,
                    )

                @pl.when(j < n_pairs - 1)
                def _next():
                    repack(h + 2, idxh)
                    pltpu.async_copy(table2.at[idxh], rows, gs)

            return carry

        lax.fori_loop(0, n_pairs, pair, 0)
        for tiles, ss in ((tiles_a, ssa), (tiles_b, ssb)):
            for dt in range(n_dtiles):
                pltpu.make_async_copy(
                    tiles.at[pl.ds(dt * tile_words, tile_words)],
                    out_hbm.at[pl.ds(out_base, tile_words)],
                    ss,
                ).wait()

    return k


def kernel(token_ids, weight):
    batch, hist = token_ids.shape
    flat_idx = token_ids.reshape(batch * hist).astype(jnp.int32)
    outbuf = _make_gather(batch, hist)(flat_idx, weight)
    out = (
        outbuf.reshape(hist, D // 8, batch // 128, 8, 128)
        .transpose(2, 4, 0, 1, 3)
        .reshape(batch, hist, D)
    )
    return out
